# baseline (speedup 1.0000x reference)
import functools

import jax
import jax.numpy as jnp
from jax import lax
from jax.experimental import pallas as pl
from jax.experimental.pallas import tpu as pltpu
from jax.experimental.pallas import tpu_sc as plsc

N_NODES = 10000
N_EDGES = 320000
D_IN = 128
D_HID = 256
D_OUT = 64

NC = 2
NS = 16
NW = NC * NS
NPAD = 10240
EB = 96
NB = 106
EPT = NB * EB
E_PAD = NW * EPT
SPT = NPAD // NS
CH = 64

_MESH = dict(core_axis_name="c", subcore_axis_name="s", num_cores=NC,
             num_subcores=NS)


@functools.cache
def _make_degree():
    return functools.partial(
        pl.kernel,
        out_type=jax.ShapeDtypeStruct((NC, NPAD), jnp.float32),
        mesh=plsc.VectorSubcoreMesh(**_MESH),
        scratch_types=[
            pltpu.VMEM((NB, EB), jnp.int32),
            pltpu.VMEM((EB,), jnp.float32),
            pltpu.VMEM((SPT,), jnp.float32),
            pltpu.VMEM_SHARED((NPAD,), jnp.float32),
        ],
    )(_sc_degree_body)


def _sc_degree_body(dst_hbm, out_hbm, idx_v, ones_v, buf_v, acc_sh):
    c = lax.axis_index("c")
    s = lax.axis_index("s")
    tile = c * NS + s
    for i in range(EB // 16):
        ones_v[pl.ds(16 * i, 16)] = jnp.ones((16,), jnp.float32)
    for i in range(SPT // 16):
        buf_v[pl.ds(16 * i, 16)] = jnp.zeros((16,), jnp.float32)
    pltpu.sync_copy(buf_v, acc_sh.at[pl.ds(s * SPT, SPT)])
    pltpu.sync_copy(dst_hbm.at[tile], idx_v)
    plsc.subcore_barrier()

    def body(b, carry):
        pltpu.sync_copy(ones_v, acc_sh.at[idx_v.at[b]], add=True)
        return carry

    lax.fori_loop(0, NB, body, 0)
    plsc.subcore_barrier()
    pltpu.sync_copy(acc_sh.at[pl.ds(s * SPT, SPT)], buf_v)
    pltpu.sync_copy(buf_v, out_hbm.at[c, pl.ds(s * SPT, SPT)])


@functools.cache
def _make_rowscatter(D):
    @functools.partial(
        pl.kernel,
        out_type=jax.ShapeDtypeStruct((NC, NPAD, D), jnp.float32),
        mesh=plsc.VectorSubcoreMesh(**_MESH),
        scratch_types=[
            pltpu.VMEM((NB, EB), jnp.int32),
            pltpu.VMEM((NB, EB), jnp.int32),
            pltpu.VMEM((EB, D), jnp.float32),
            pltpu.VMEM((EB, D), jnp.float32),
            pltpu.SemaphoreType.DMA,
            pltpu.SemaphoreType.DMA,
            pltpu.VMEM_SHARED((NPAD, D), jnp.float32),
        ],
        compiler_params=pltpu.CompilerParams(use_tc_tiling_on_sc=False),
    )
    def rowscatter(table_hbm, src_hbm, dst_hbm, out_hbm, isv, idv, rows0, rows1,
                   sem0, sem1, acc_sh):
        c = lax.axis_index("c")
        s = lax.axis_index("s")
        tile = c * NS + s
        pltpu.sync_copy(src_hbm.at[tile], isv)
        pltpu.sync_copy(dst_hbm.at[tile], idv)
        for kk in range(SPT // CH):
            r0 = s * SPT + kk * CH
            pltpu.sync_copy(table_hbm.at[pl.ds(r0, CH)], rows0.at[pl.ds(0, CH)])
            pltpu.sync_copy(rows0.at[pl.ds(0, CH)], acc_sh.at[pl.ds(r0, CH)])
        plsc.subcore_barrier()

        def body(b, carry):
            pltpu.sync_copy(table_hbm.at[isv.at[b]], rows0)
            pltpu.sync_copy(rows0, acc_sh.at[idv.at[b]], add=True)
            return carry

        lax.fori_loop(0, NB, body, 0)
        plsc.subcore_barrier()
        for kk in range(SPT // CH):
            r0 = s * SPT + kk * CH
            pltpu.sync_copy(acc_sh.at[pl.ds(r0, CH)], rows0.at[pl.ds(0, CH)])
            pltpu.sync_copy(rows0.at[pl.ds(0, CH)], out_hbm.at[c, pl.ds(r0, CH)])

    return rowscatter


def _tc_prescale_body(deg_ref, x_ref, dis_ref, xs_ref):
    deg = deg_ref[:, 0:1] + deg_ref[:, 1:2] + 1.0
    dis = lax.rsqrt(deg)
    dis_b = jnp.broadcast_to(dis, dis_ref.shape)
    dis_ref[...] = dis_b
    xs_ref[...] = x_ref[...] * dis_b


_RB = 1280
_NRB = NPAD // _RB


def _tc_prescale(deg2t, xpad):
    return pl.pallas_call(
        _tc_prescale_body,
        grid=(_NRB,),
        in_specs=[
            pl.BlockSpec((_RB, NC), lambda i: (i, 0)),
            pl.BlockSpec((_RB, D_IN), lambda i: (i, 0)),
        ],
        out_specs=[
            pl.BlockSpec((_RB, D_IN), lambda i: (i, 0)),
            pl.BlockSpec((_RB, D_IN), lambda i: (i, 0)),
        ],
        out_shape=[
            jax.ShapeDtypeStruct((NPAD, D_IN), jnp.float32),
            jax.ShapeDtypeStruct((NPAD, D_IN), jnp.float32),
        ],
    )(deg2t, xpad)


def _tc_mid_body(acc_ref, xs_ref, dis_ref, w1_ref, b1_ref, w2_ref, ts_ref):
    agg = (acc_ref[0] + acc_ref[1] - xs_ref[...]) * dis_ref[...]
    h = jnp.dot(agg, w1_ref[...], preferred_element_type=jnp.float32)
    h = jnp.maximum(h + b1_ref[...], 0.0)
    t = jnp.dot(h, w2_ref[...], preferred_element_type=jnp.float32)
    ts_ref[...] = t * dis_ref[:, :D_OUT]


def _tc_mid(acc, xs, dis128, W1, b1r, W2):
    return pl.pallas_call(
        _tc_mid_body,
        grid=(_NRB,),
        in_specs=[
            pl.BlockSpec((NC, _RB, D_IN), lambda i: (0, i, 0)),
            pl.BlockSpec((_RB, D_IN), lambda i: (i, 0)),
            pl.BlockSpec((_RB, D_IN), lambda i: (i, 0)),
            pl.BlockSpec((D_IN, D_HID), lambda i: (0, 0)),
            pl.BlockSpec((1, D_HID), lambda i: (0, 0)),
            pl.BlockSpec((D_HID, D_OUT), lambda i: (0, 0)),
        ],
        out_specs=pl.BlockSpec((_RB, D_OUT), lambda i: (i, 0)),
        out_shape=jax.ShapeDtypeStruct((NPAD, D_OUT), jnp.float32),
    )(acc, xs, dis128, W1, b1r, W2)


def _tc_final_body(acc_ref, ts_ref, dis_ref, b2_ref, out_ref):
    agg = (acc_ref[0] + acc_ref[1] - ts_ref[...]) * dis_ref[:, :D_OUT]
    out_ref[...] = agg + b2_ref[...]


def _tc_final(acc2, ts, dis128, b2r):
    return pl.pallas_call(
        _tc_final_body,
        grid=(_NRB,),
        in_specs=[
            pl.BlockSpec((NC, _RB, D_OUT), lambda i: (0, i, 0)),
            pl.BlockSpec((_RB, D_OUT), lambda i: (i, 0)),
            pl.BlockSpec((_RB, D_IN), lambda i: (i, 0)),
            pl.BlockSpec((1, D_OUT), lambda i: (0, 0)),
        ],
        out_specs=pl.BlockSpec((_RB, D_OUT), lambda i: (i, 0)),
        out_shape=jax.ShapeDtypeStruct((NPAD, D_OUT), jnp.float32),
    )(acc2, ts, dis128, b2r)


def kernel(x, edge_index, W1, b1, W2, b2):
    x = x.astype(jnp.float32)
    ei = jnp.pad(
        edge_index.astype(jnp.int32),
        ((0, 0), (0, E_PAD - N_EDGES)),
        constant_values=NPAD - 1,
    )
    src = ei[0].reshape(NW, NB, EB)
    dst = ei[1].reshape(NW, NB, EB)
    xpad = jnp.pad(x, ((0, NPAD - N_NODES), (0, 0)))

    deg2 = _make_degree()(dst)
    dis128, xs = _tc_prescale(deg2.T, xpad)
    acc1 = _make_rowscatter(D_IN)(xs, src, dst)
    ts = _tc_mid(acc1, xs, dis128, W1, b1.reshape(1, D_HID), W2)
    acc2 = _make_rowscatter(D_OUT)(ts, src, dst)
    out = _tc_final(acc2, ts, dis128, b2.reshape(1, D_OUT))
    return out[:N_NODES]


# speedup vs baseline: 2.8489x; 2.8489x over previous
import functools

import jax
import jax.numpy as jnp
from jax import lax
from jax.experimental import pallas as pl
from jax.experimental.pallas import tpu as pltpu
from jax.experimental.pallas import tpu_sc as plsc

N_NODES = 10000
N_EDGES = 320000
D_IN = 128
D_HID = 256
D_OUT = 64

NC = 2
NS = 16
NW = NC * NS
NPAD = 10240
EB = 96
NB = 106
EPT = NB * EB
E_PAD = NW * EPT
SPT = NPAD // NS
CH = 64

_MESH = dict(core_axis_name="c", subcore_axis_name="s", num_cores=NC,
             num_subcores=NS)


@functools.cache
def _make_degree():
    return functools.partial(
        pl.kernel,
        out_type=jax.ShapeDtypeStruct((NC, NPAD), jnp.float32),
        mesh=plsc.VectorSubcoreMesh(**_MESH),
        scratch_types=[
            pltpu.VMEM((NB, EB), jnp.int32),
            pltpu.VMEM((EB,), jnp.float32),
            pltpu.VMEM((SPT,), jnp.float32),
            pltpu.VMEM_SHARED((NPAD,), jnp.float32),
        ],
    )(_sc_degree_body)


def _sc_degree_body(dst_hbm, out_hbm, idx_v, ones_v, buf_v, acc_sh):
    c = lax.axis_index("c")
    s = lax.axis_index("s")
    tile = c * NS + s
    for i in range(EB // 16):
        ones_v[pl.ds(16 * i, 16)] = jnp.ones((16,), jnp.float32)
    for i in range(SPT // 16):
        buf_v[pl.ds(16 * i, 16)] = jnp.zeros((16,), jnp.float32)
    pltpu.sync_copy(buf_v, acc_sh.at[pl.ds(s * SPT, SPT)])
    pltpu.sync_copy(dst_hbm.at[tile], idx_v)
    plsc.subcore_barrier()

    def body(b, carry):
        pltpu.sync_copy(ones_v, acc_sh.at[idx_v.at[b]], add=True)
        return carry

    lax.fori_loop(0, NB, body, 0)
    plsc.subcore_barrier()
    pltpu.sync_copy(acc_sh.at[pl.ds(s * SPT, SPT)], buf_v)
    pltpu.sync_copy(buf_v, out_hbm.at[c, pl.ds(s * SPT, SPT)])


@functools.cache
def _make_rowscatter(D):
    @functools.partial(
        pl.kernel,
        out_type=jax.ShapeDtypeStruct((NC, NPAD, D), jnp.float32),
        mesh=plsc.VectorSubcoreMesh(**_MESH),
        scratch_types=[
            pltpu.VMEM((NB, EB), jnp.int32),
            pltpu.VMEM((NB, EB), jnp.int32),
            pltpu.VMEM((EB, D), jnp.float32),
            pltpu.VMEM((EB, D), jnp.float32),
            pltpu.SemaphoreType.DMA,
            pltpu.SemaphoreType.DMA,
            pltpu.VMEM_SHARED((NPAD, D), jnp.float32),
        ],
        compiler_params=pltpu.CompilerParams(use_tc_tiling_on_sc=False),
    )
    def rowscatter(table_hbm, src_hbm, dst_hbm, out_hbm, isv, idv, rows0, rows1,
                   sem0, sem1, acc_sh):
        c = lax.axis_index("c")
        s = lax.axis_index("s")
        tile = c * NS + s
        pltpu.sync_copy(src_hbm.at[tile], isv)
        pltpu.sync_copy(dst_hbm.at[tile], idv)
        for kk in range(SPT // CH):
            r0 = s * SPT + kk * CH
            pltpu.sync_copy(table_hbm.at[pl.ds(r0, CH)], rows0.at[pl.ds(0, CH)])
            pltpu.sync_copy(rows0.at[pl.ds(0, CH)], acc_sh.at[pl.ds(r0, CH)])
        plsc.subcore_barrier()

        pltpu.async_copy(table_hbm.at[isv.at[0]], rows0, sem0)
        pltpu.async_copy(table_hbm.at[isv.at[1]], rows1, sem1)

        def body(i, carry):
            b0 = 2 * i
            b1 = b0 + 1
            n0 = jnp.where(b0 + 2 >= NB, 0, b0 + 2)
            n1 = jnp.where(b1 + 2 >= NB, 1, b1 + 2)
            pltpu.make_async_copy(table_hbm.at[isv.at[b0]], rows0, sem0).wait()
            pltpu.sync_copy(rows0, acc_sh.at[idv.at[b0]], add=True)
            pltpu.async_copy(table_hbm.at[isv.at[n0]], rows0, sem0)
            pltpu.make_async_copy(table_hbm.at[isv.at[b1]], rows1, sem1).wait()
            pltpu.sync_copy(rows1, acc_sh.at[idv.at[b1]], add=True)
            pltpu.async_copy(table_hbm.at[isv.at[n1]], rows1, sem1)
            return carry

        lax.fori_loop(0, NB // 2, body, 0)
        pltpu.make_async_copy(table_hbm.at[isv.at[0]], rows0, sem0).wait()
        pltpu.make_async_copy(table_hbm.at[isv.at[1]], rows1, sem1).wait()
        plsc.subcore_barrier()
        for kk in range(SPT // CH):
            r0 = s * SPT + kk * CH
            pltpu.sync_copy(acc_sh.at[pl.ds(r0, CH)], rows0.at[pl.ds(0, CH)])
            pltpu.sync_copy(rows0.at[pl.ds(0, CH)], out_hbm.at[c, pl.ds(r0, CH)])

    return rowscatter


def _tc_prescale_body(deg_ref, x_ref, dis_ref, xs_ref):
    deg = deg_ref[:, 0:1] + deg_ref[:, 1:2] + 1.0
    dis = lax.rsqrt(deg)
    dis_b = jnp.broadcast_to(dis, dis_ref.shape)
    dis_ref[...] = dis_b
    xs_ref[...] = x_ref[...] * dis_b


_RB = 1280
_NRB = NPAD // _RB


def _tc_prescale(deg2t, xpad):
    return pl.pallas_call(
        _tc_prescale_body,
        grid=(_NRB,),
        in_specs=[
            pl.BlockSpec((_RB, NC), lambda i: (i, 0)),
            pl.BlockSpec((_RB, D_IN), lambda i: (i, 0)),
        ],
        out_specs=[
            pl.BlockSpec((_RB, D_IN), lambda i: (i, 0)),
            pl.BlockSpec((_RB, D_IN), lambda i: (i, 0)),
        ],
        out_shape=[
            jax.ShapeDtypeStruct((NPAD, D_IN), jnp.float32),
            jax.ShapeDtypeStruct((NPAD, D_IN), jnp.float32),
        ],
    )(deg2t, xpad)


def _tc_mid_body(acc_ref, xs_ref, dis_ref, w1_ref, b1_ref, w2_ref, ts_ref):
    agg = (acc_ref[0] + acc_ref[1] - xs_ref[...]) * dis_ref[...]
    h = jnp.dot(agg, w1_ref[...], preferred_element_type=jnp.float32)
    h = jnp.maximum(h + b1_ref[...], 0.0)
    t = jnp.dot(h, w2_ref[...], preferred_element_type=jnp.float32)
    ts_ref[...] = t * dis_ref[:, :D_OUT]


def _tc_mid(acc, xs, dis128, W1, b1r, W2):
    return pl.pallas_call(
        _tc_mid_body,
        grid=(_NRB,),
        in_specs=[
            pl.BlockSpec((NC, _RB, D_IN), lambda i: (0, i, 0)),
            pl.BlockSpec((_RB, D_IN), lambda i: (i, 0)),
            pl.BlockSpec((_RB, D_IN), lambda i: (i, 0)),
            pl.BlockSpec((D_IN, D_HID), lambda i: (0, 0)),
            pl.BlockSpec((1, D_HID), lambda i: (0, 0)),
            pl.BlockSpec((D_HID, D_OUT), lambda i: (0, 0)),
        ],
        out_specs=pl.BlockSpec((_RB, D_OUT), lambda i: (i, 0)),
        out_shape=jax.ShapeDtypeStruct((NPAD, D_OUT), jnp.float32),
    )(acc, xs, dis128, W1, b1r, W2)


def _tc_final_body(acc_ref, ts_ref, dis_ref, b2_ref, out_ref):
    agg = (acc_ref[0] + acc_ref[1] - ts_ref[...]) * dis_ref[:, :D_OUT]
    out_ref[...] = agg + b2_ref[...]


def _tc_final(acc2, ts, dis128, b2r):
    return pl.pallas_call(
        _tc_final_body,
        grid=(_NRB,),
        in_specs=[
            pl.BlockSpec((NC, _RB, D_OUT), lambda i: (0, i, 0)),
            pl.BlockSpec((_RB, D_OUT), lambda i: (i, 0)),
            pl.BlockSpec((_RB, D_IN), lambda i: (i, 0)),
            pl.BlockSpec((1, D_OUT), lambda i: (0, 0)),
        ],
        out_specs=pl.BlockSpec((_RB, D_OUT), lambda i: (i, 0)),
        out_shape=jax.ShapeDtypeStruct((NPAD, D_OUT), jnp.float32),
    )(acc2, ts, dis128, b2r)


def kernel(x, edge_index, W1, b1, W2, b2):
    x = x.astype(jnp.float32)
    pad_idx = N_NODES + (
        jnp.arange(E_PAD - N_EDGES, dtype=jnp.int32) % (NPAD - N_NODES)
    )
    src = jnp.concatenate([edge_index[0].astype(jnp.int32), pad_idx])
    dst = jnp.concatenate([edge_index[1].astype(jnp.int32), pad_idx])
    src = src.reshape(NW, NB, EB)
    dst = dst.reshape(NW, NB, EB)
    xpad = jnp.pad(x, ((0, NPAD - N_NODES), (0, 0)))

    deg2 = _make_degree()(dst)
    dis128, xs = _tc_prescale(deg2.T, xpad)
    acc1 = _make_rowscatter(D_IN)(xs, src, dst)
    ts = _tc_mid(acc1, xs, dis128, W1, b1.reshape(1, D_HID), W2)
    acc2 = _make_rowscatter(D_OUT)(ts, src, dst)
    out = _tc_final(acc2, ts, dis128, b2.reshape(1, D_OUT))
    return out[:N_NODES]


# speedup vs baseline: 2.9585x; 1.0385x over previous
import functools

import jax
import jax.numpy as jnp
from jax import lax
from jax.experimental import pallas as pl
from jax.experimental.pallas import tpu as pltpu
from jax.experimental.pallas import tpu_sc as plsc

N_NODES = 10000
N_EDGES = 320000
D_IN = 128
D_HID = 256
D_OUT = 64

NC = 2
NS = 16
NW = NC * NS
NPAD = 10240
EB1, NB1 = 96, 106
EB2, NB2 = 128, 80
SPT = NPAD // NS
CH = 64

_MESH = dict(core_axis_name="c", subcore_axis_name="s", num_cores=NC,
             num_subcores=NS)


@functools.cache
def _make_degree():
    @functools.partial(
        pl.kernel,
        out_type=jax.ShapeDtypeStruct((NC, NPAD), jnp.float32),
        mesh=plsc.VectorSubcoreMesh(**_MESH),
        scratch_types=[
            pltpu.VMEM((NB2, EB2), jnp.int32),
            pltpu.VMEM((EB2,), jnp.float32),
            pltpu.VMEM((SPT,), jnp.float32),
            pltpu.VMEM_SHARED((NPAD,), jnp.float32),
        ],
    )
    def degree(dst_hbm, out_hbm, idx_v, ones_v, buf_v, acc_sh):
        c = lax.axis_index("c")
        s = lax.axis_index("s")
        tile = c * NS + s
        for i in range(EB2 // 16):
            ones_v[pl.ds(16 * i, 16)] = jnp.ones((16,), jnp.float32)
        for i in range(SPT // 16):
            buf_v[pl.ds(16 * i, 16)] = jnp.zeros((16,), jnp.float32)
        pltpu.sync_copy(buf_v, acc_sh.at[pl.ds(s * SPT, SPT)])
        pltpu.sync_copy(dst_hbm.at[tile], idx_v)
        plsc.subcore_barrier()

        def body(b, carry):
            pltpu.sync_copy(ones_v, acc_sh.at[idx_v.at[b]], add=True)
            return carry

        lax.fori_loop(0, NB2, body, 0)
        plsc.subcore_barrier()
        pltpu.sync_copy(acc_sh.at[pl.ds(s * SPT, SPT)], buf_v)
        pltpu.sync_copy(buf_v, out_hbm.at[c, pl.ds(s * SPT, SPT)])

    return degree


@functools.cache
def _make_rowscatter(D, EB, NB):
    @functools.partial(
        pl.kernel,
        out_type=jax.ShapeDtypeStruct((NC, NPAD, D), jnp.float32),
        mesh=plsc.VectorSubcoreMesh(**_MESH),
        scratch_types=[
            pltpu.VMEM((NB, EB), jnp.int32),
            pltpu.VMEM((NB, EB), jnp.int32),
            pltpu.VMEM((EB, D), jnp.float32),
            pltpu.VMEM((EB, D), jnp.float32),
            pltpu.SemaphoreType.DMA,
            pltpu.SemaphoreType.DMA,
            pltpu.VMEM_SHARED((NPAD, D), jnp.float32),
        ],
        compiler_params=pltpu.CompilerParams(use_tc_tiling_on_sc=False),
    )
    def rowscatter(table_hbm, src_hbm, dst_hbm, out_hbm, isv, idv, rows0, rows1,
                   sem0, sem1, acc_sh):
        c = lax.axis_index("c")
        s = lax.axis_index("s")
        tile = c * NS + s
        pltpu.sync_copy(src_hbm.at[tile], isv)
        pltpu.sync_copy(dst_hbm.at[tile], idv)
        for kk in range(SPT // CH):
            r0 = s * SPT + kk * CH
            pltpu.sync_copy(table_hbm.at[pl.ds(r0, CH)], rows0.at[pl.ds(0, CH)])
            pltpu.sync_copy(rows0.at[pl.ds(0, CH)], acc_sh.at[pl.ds(r0, CH)])
        plsc.subcore_barrier()

        pltpu.async_copy(table_hbm.at[isv.at[0]], rows0, sem0)
        pltpu.async_copy(table_hbm.at[isv.at[1]], rows1, sem1)

        def body(i, carry):
            b0 = 2 * i
            b1 = b0 + 1
            n0 = jnp.where(b0 + 2 >= NB, 0, b0 + 2)
            n1 = jnp.where(b1 + 2 >= NB, 1, b1 + 2)
            pltpu.make_async_copy(table_hbm.at[isv.at[b0]], rows0, sem0).wait()
            pltpu.sync_copy(rows0, acc_sh.at[idv.at[b0]], add=True)
            pltpu.async_copy(table_hbm.at[isv.at[n0]], rows0, sem0)
            pltpu.make_async_copy(table_hbm.at[isv.at[b1]], rows1, sem1).wait()
            pltpu.sync_copy(rows1, acc_sh.at[idv.at[b1]], add=True)
            pltpu.async_copy(table_hbm.at[isv.at[n1]], rows1, sem1)
            return carry

        lax.fori_loop(0, NB // 2, body, 0)
        pltpu.make_async_copy(table_hbm.at[isv.at[0]], rows0, sem0).wait()
        pltpu.make_async_copy(table_hbm.at[isv.at[1]], rows1, sem1).wait()
        plsc.subcore_barrier()
        for kk in range(SPT // CH):
            r0 = s * SPT + kk * CH
            pltpu.sync_copy(acc_sh.at[pl.ds(r0, CH)], rows0.at[pl.ds(0, CH)])
            pltpu.sync_copy(rows0.at[pl.ds(0, CH)], out_hbm.at[c, pl.ds(r0, CH)])

    return rowscatter


def _tc_prescale_body(deg_ref, x_ref, dis_ref, xs_ref):
    deg = deg_ref[:, 0:1] + deg_ref[:, 1:2] + 1.0
    dis = lax.rsqrt(deg)
    dis_b = jnp.broadcast_to(dis, dis_ref.shape)
    dis_ref[...] = dis_b
    xs_ref[...] = x_ref[...] * dis_b


_RB = 1280
_NRB = NPAD // _RB


def _tc_prescale(deg2t, xpad):
    return pl.pallas_call(
        _tc_prescale_body,
        grid=(_NRB,),
        in_specs=[
            pl.BlockSpec((_RB, NC), lambda i: (i, 0)),
            pl.BlockSpec((_RB, D_IN), lambda i: (i, 0)),
        ],
        out_specs=[
            pl.BlockSpec((_RB, D_IN), lambda i: (i, 0)),
            pl.BlockSpec((_RB, D_IN), lambda i: (i, 0)),
        ],
        out_shape=[
            jax.ShapeDtypeStruct((NPAD, D_IN), jnp.float32),
            jax.ShapeDtypeStruct((NPAD, D_IN), jnp.float32),
        ],
    )(deg2t, xpad)


def _tc_mid_body(acc_ref, xs_ref, dis_ref, w1_ref, b1_ref, w2_ref, ts_ref):
    agg = (acc_ref[0] + acc_ref[1] - xs_ref[...]) * dis_ref[...]
    h = jnp.dot(agg, w1_ref[...], preferred_element_type=jnp.float32)
    h = jnp.maximum(h + b1_ref[...], 0.0)
    t = jnp.dot(h, w2_ref[...], preferred_element_type=jnp.float32)
    ts_ref[...] = t * dis_ref[:, :D_OUT]


def _tc_mid(acc, xs, dis128, W1, b1r, W2):
    return pl.pallas_call(
        _tc_mid_body,
        grid=(_NRB,),
        in_specs=[
            pl.BlockSpec((NC, _RB, D_IN), lambda i: (0, i, 0)),
            pl.BlockSpec((_RB, D_IN), lambda i: (i, 0)),
            pl.BlockSpec((_RB, D_IN), lambda i: (i, 0)),
            pl.BlockSpec((D_IN, D_HID), lambda i: (0, 0)),
            pl.BlockSpec((1, D_HID), lambda i: (0, 0)),
            pl.BlockSpec((D_HID, D_OUT), lambda i: (0, 0)),
        ],
        out_specs=pl.BlockSpec((_RB, D_OUT), lambda i: (i, 0)),
        out_shape=jax.ShapeDtypeStruct((NPAD, D_OUT), jnp.float32),
    )(acc, xs, dis128, W1, b1r, W2)


def _tc_final_body(acc_ref, ts_ref, dis_ref, b2_ref, out_ref):
    agg = (acc_ref[0] + acc_ref[1] - ts_ref[...]) * dis_ref[:, :D_OUT]
    out_ref[...] = agg + b2_ref[...]


def _tc_final(acc2, ts, dis128, b2r):
    return pl.pallas_call(
        _tc_final_body,
        grid=(_NRB,),
        in_specs=[
            pl.BlockSpec((NC, _RB, D_OUT), lambda i: (0, i, 0)),
            pl.BlockSpec((_RB, D_OUT), lambda i: (i, 0)),
            pl.BlockSpec((_RB, D_IN), lambda i: (i, 0)),
            pl.BlockSpec((1, D_OUT), lambda i: (0, 0)),
        ],
        out_specs=pl.BlockSpec((_RB, D_OUT), lambda i: (i, 0)),
        out_shape=jax.ShapeDtypeStruct((NPAD, D_OUT), jnp.float32),
    )(acc2, ts, dis128, b2r)


def kernel(x, edge_index, W1, b1, W2, b2):
    x = x.astype(jnp.float32)

    def edge_layout(eb, nb):
        e_pad = NW * nb * eb
        pad_idx = N_NODES + (
            jnp.arange(e_pad - N_EDGES, dtype=jnp.int32) % (NPAD - N_NODES)
        )
        s = jnp.concatenate([edge_index[0].astype(jnp.int32), pad_idx])
        d = jnp.concatenate([edge_index[1].astype(jnp.int32), pad_idx])
        return s.reshape(NW, nb, eb), d.reshape(NW, nb, eb)

    src1, dst1 = edge_layout(EB1, NB1)
    src2, dst2 = edge_layout(EB2, NB2)
    xpad = jnp.pad(x, ((0, NPAD - N_NODES), (0, 0)))

    deg2 = _make_degree()(dst2)
    dis128, xs = _tc_prescale(deg2.T, xpad)
    acc1 = _make_rowscatter(D_IN, EB1, NB1)(xs, src1, dst1)
    ts = _tc_mid(acc1, xs, dis128, W1, b1.reshape(1, D_HID), W2)
    acc2 = _make_rowscatter(D_OUT, EB2, NB2)(ts, src2, dst2)
    out = _tc_final(acc2, ts, dis128, b2.reshape(1, D_OUT))
    return out[:N_NODES]


# speedup vs baseline: 3.1133x; 1.0523x over previous
import functools

import jax
import jax.numpy as jnp
from jax import lax
from jax.experimental import pallas as pl
from jax.experimental.pallas import tpu as pltpu
from jax.experimental.pallas import tpu_sc as plsc

N_NODES = 10000
N_EDGES = 320000
D_IN = 128
D_HID = 256
D_OUT = 64

NC = 2
NS = 16
NW = NC * NS
NPAD = 10240
EB1, NB1 = 96, 106
EB2, NB2 = 128, 80
SPT = NPAD // NS
CH = 64

_MESH = dict(core_axis_name="c", subcore_axis_name="s", num_cores=NC,
             num_subcores=NS)


@functools.cache
def _make_degree():
    @functools.partial(
        pl.kernel,
        out_type=jax.ShapeDtypeStruct((NC, NPAD), jnp.float32),
        mesh=plsc.VectorSubcoreMesh(**_MESH),
        scratch_types=[
            pltpu.VMEM((NB2, EB2), jnp.int32),
            pltpu.VMEM((EB2,), jnp.float32),
            pltpu.VMEM((SPT,), jnp.float32),
            pltpu.VMEM_SHARED((NPAD,), jnp.float32),
        ],
    )
    def degree(dst_hbm, out_hbm, idx_v, ones_v, buf_v, acc_sh):
        c = lax.axis_index("c")
        s = lax.axis_index("s")
        tile = c * NS + s
        for i in range(EB2 // 16):
            ones_v[pl.ds(16 * i, 16)] = jnp.ones((16,), jnp.float32)
        for i in range(SPT // 16):
            buf_v[pl.ds(16 * i, 16)] = jnp.zeros((16,), jnp.float32)
        pltpu.sync_copy(buf_v, acc_sh.at[pl.ds(s * SPT, SPT)])
        pltpu.sync_copy(dst_hbm.at[tile], idx_v)
        plsc.subcore_barrier()

        def body(b, carry):
            pltpu.sync_copy(ones_v, acc_sh.at[idx_v.at[b]], add=True)
            return carry

        lax.fori_loop(0, NB2, body, 0)
        plsc.subcore_barrier()
        pltpu.sync_copy(acc_sh.at[pl.ds(s * SPT, SPT)], buf_v)
        pltpu.sync_copy(buf_v, out_hbm.at[c, pl.ds(s * SPT, SPT)])

    return degree


@functools.cache
def _make_rowscatter(D, EB, NB):
    @functools.partial(
        pl.kernel,
        out_type=jax.ShapeDtypeStruct((NC, NPAD, D), jnp.float32),
        mesh=plsc.VectorSubcoreMesh(**_MESH),
        scratch_types=[
            pltpu.VMEM((NB, EB), jnp.int32),
            pltpu.VMEM((NB, EB), jnp.int32),
            pltpu.VMEM((EB, D), jnp.float32),
            pltpu.VMEM((EB, D), jnp.float32),
            pltpu.SemaphoreType.DMA,
            pltpu.SemaphoreType.DMA,
            pltpu.VMEM_SHARED((NPAD, D), jnp.float32),
        ],
        compiler_params=pltpu.CompilerParams(use_tc_tiling_on_sc=False),
    )
    def rowscatter(table_hbm, src_hbm, dst_hbm, out_hbm, isv, idv, rows0, rows1,
                   sem0, sem1, acc_sh):
        c = lax.axis_index("c")
        s = lax.axis_index("s")
        tile = c * NS + s
        pltpu.sync_copy(src_hbm.at[tile], isv)
        pltpu.sync_copy(dst_hbm.at[tile], idv)
        pltpu.sync_copy(table_hbm.at[pl.ds(s * SPT, SPT)],
                        acc_sh.at[pl.ds(s * SPT, SPT)])
        plsc.subcore_barrier()

        pltpu.async_copy(table_hbm.at[isv.at[0]], rows0, sem0)
        pltpu.async_copy(table_hbm.at[isv.at[1]], rows1, sem1)

        def body(i, carry):
            b0 = 2 * i
            b1 = b0 + 1
            n0 = jnp.where(b0 + 2 >= NB, 0, b0 + 2)
            n1 = jnp.where(b1 + 2 >= NB, 1, b1 + 2)
            pltpu.make_async_copy(table_hbm.at[isv.at[b0]], rows0, sem0).wait()
            pltpu.sync_copy(rows0, acc_sh.at[idv.at[b0]], add=True)
            pltpu.async_copy(table_hbm.at[isv.at[n0]], rows0, sem0)
            pltpu.make_async_copy(table_hbm.at[isv.at[b1]], rows1, sem1).wait()
            pltpu.sync_copy(rows1, acc_sh.at[idv.at[b1]], add=True)
            pltpu.async_copy(table_hbm.at[isv.at[n1]], rows1, sem1)
            return carry

        lax.fori_loop(0, NB // 2, body, 0)
        pltpu.make_async_copy(table_hbm.at[isv.at[0]], rows0, sem0).wait()
        pltpu.make_async_copy(table_hbm.at[isv.at[1]], rows1, sem1).wait()
        plsc.subcore_barrier()
        pltpu.sync_copy(acc_sh.at[pl.ds(s * SPT, SPT)],
                        out_hbm.at[c, pl.ds(s * SPT, SPT)])

    return rowscatter


def _tc_prescale_body(deg_ref, x_ref, dis_ref, xs_ref):
    deg = deg_ref[:, 0:1] + deg_ref[:, 1:2] + 1.0
    dis = lax.rsqrt(deg)
    dis_b = jnp.broadcast_to(dis, dis_ref.shape)
    dis_ref[...] = dis_b
    xs_ref[...] = x_ref[...] * dis_b


_RB = 1280
_NRB = NPAD // _RB


def _tc_prescale(deg2t, xpad):
    return pl.pallas_call(
        _tc_prescale_body,
        grid=(_NRB,),
        in_specs=[
            pl.BlockSpec((_RB, NC), lambda i: (i, 0)),
            pl.BlockSpec((_RB, D_IN), lambda i: (i, 0)),
        ],
        out_specs=[
            pl.BlockSpec((_RB, D_IN), lambda i: (i, 0)),
            pl.BlockSpec((_RB, D_IN), lambda i: (i, 0)),
        ],
        out_shape=[
            jax.ShapeDtypeStruct((NPAD, D_IN), jnp.float32),
            jax.ShapeDtypeStruct((NPAD, D_IN), jnp.float32),
        ],
    )(deg2t, xpad)


def _tc_mid_body(acc_ref, xs_ref, dis_ref, w1_ref, b1_ref, w2_ref, ts_ref):
    agg = (acc_ref[0] + acc_ref[1] - xs_ref[...]) * dis_ref[...]
    h = jnp.dot(agg, w1_ref[...], preferred_element_type=jnp.float32)
    h = jnp.maximum(h + b1_ref[...], 0.0)
    t = jnp.dot(h, w2_ref[...], preferred_element_type=jnp.float32)
    ts_ref[...] = t * dis_ref[:, :D_OUT]


def _tc_mid(acc, xs, dis128, W1, b1r, W2):
    return pl.pallas_call(
        _tc_mid_body,
        grid=(_NRB,),
        in_specs=[
            pl.BlockSpec((NC, _RB, D_IN), lambda i: (0, i, 0)),
            pl.BlockSpec((_RB, D_IN), lambda i: (i, 0)),
            pl.BlockSpec((_RB, D_IN), lambda i: (i, 0)),
            pl.BlockSpec((D_IN, D_HID), lambda i: (0, 0)),
            pl.BlockSpec((1, D_HID), lambda i: (0, 0)),
            pl.BlockSpec((D_HID, D_OUT), lambda i: (0, 0)),
        ],
        out_specs=pl.BlockSpec((_RB, D_OUT), lambda i: (i, 0)),
        out_shape=jax.ShapeDtypeStruct((NPAD, D_OUT), jnp.float32),
    )(acc, xs, dis128, W1, b1r, W2)


def _tc_final_body(acc_ref, ts_ref, dis_ref, b2_ref, out_ref):
    agg = (acc_ref[0] + acc_ref[1] - ts_ref[...]) * dis_ref[:, :D_OUT]
    out_ref[...] = agg + b2_ref[...]


def _tc_final(acc2, ts, dis128, b2r):
    return pl.pallas_call(
        _tc_final_body,
        grid=(_NRB,),
        in_specs=[
            pl.BlockSpec((NC, _RB, D_OUT), lambda i: (0, i, 0)),
            pl.BlockSpec((_RB, D_OUT), lambda i: (i, 0)),
            pl.BlockSpec((_RB, D_IN), lambda i: (i, 0)),
            pl.BlockSpec((1, D_OUT), lambda i: (0, 0)),
        ],
        out_specs=pl.BlockSpec((_RB, D_OUT), lambda i: (i, 0)),
        out_shape=jax.ShapeDtypeStruct((NPAD, D_OUT), jnp.float32),
    )(acc2, ts, dis128, b2r)


def kernel(x, edge_index, W1, b1, W2, b2):
    x = x.astype(jnp.float32)

    def edge_layout(eb, nb):
        e_pad = NW * nb * eb
        pad_idx = N_NODES + (
            jnp.arange(e_pad - N_EDGES, dtype=jnp.int32) % (NPAD - N_NODES)
        )
        s = jnp.concatenate([edge_index[0].astype(jnp.int32), pad_idx])
        d = jnp.concatenate([edge_index[1].astype(jnp.int32), pad_idx])
        return s.reshape(NW, nb, eb), d.reshape(NW, nb, eb)

    src1, dst1 = edge_layout(EB1, NB1)
    src2, dst2 = edge_layout(EB2, NB2)
    xpad = jnp.pad(x, ((0, NPAD - N_NODES), (0, 0)))

    deg2 = _make_degree()(dst2)
    dis128, xs = _tc_prescale(deg2.T, xpad)
    acc1 = _make_rowscatter(D_IN, EB1, NB1)(xs, src1, dst1)
    ts = _tc_mid(acc1, xs, dis128, W1, b1.reshape(1, D_HID), W2)
    acc2 = _make_rowscatter(D_OUT, EB2, NB2)(ts, src2, dst2)
    out = _tc_final(acc2, ts, dis128, b2.reshape(1, D_OUT))
    return out[:N_NODES]
